# baseline (speedup 1.0000x reference)
import functools

import jax
import jax.numpy as jnp
from jax import lax
from jax.experimental import pallas as pl
from jax.experimental.pallas import tpu as pltpu
from jax.experimental.pallas import tpu_sc as plsc

N_NODES = 100000
IN_CH = 128
HID = 32
OUT_CH = 128
B = 8192
NS0 = 15
NS1 = 5

NC = 2
NSUB = 16
NW = NC * NSUB
TW = B // NW
CH = 16
NCHUNK = TW // CH
G0 = CH * NS0
G1 = CH * NS0 * NS1
GSUB = 120

L = 16


def _tsum(vs):
    while len(vs) > 1:
        vs = [vs[i] + vs[i + 1] if i + 1 < len(vs) else vs[i]
              for i in range(0, len(vs), 2)]
    return vs[0]


def _proj_body(x_ref, w_ref, p_ref):
    p_ref[...] = jnp.dot(x_ref[...], w_ref[...],
                         preferred_element_type=jnp.float32)


def _out_body(hs_ref, w_ref, b_ref, o_ref):
    o_ref[...] = (
        jnp.dot(hs_ref[...], w_ref[...], preferred_element_type=jnp.float32)
        + b_ref[...]
    )


def _sc_body(t_hbm, nodes_hbm, n0_hbm, n1_hbm, b0_hbm, hs_hbm,
             idxn_v, idx0a_v, idx0n_v, idx1_v, asel_v, a0_v, n0_v, n1_v,
             b0_v, hs_v, sem):
    wid = lax.axis_index("s") * NC + lax.axis_index("c")

    pltpu.sync_copy(b0_hbm, b0_v)
    b0a = b0_v[pl.ds(0, L)]
    b0b = b0_v[pl.ds(L, L)]
    b0c = b0_v[pl.ds(2 * L, L)]
    b0d = b0_v[pl.ds(3 * L, L)]
    zero = jnp.zeros((L,), jnp.float32)

    def stage_fire(ch, q):
        t0 = wid * TW + ch * CH
        pltpu.sync_copy(nodes_hbm.at[pl.ds(pl.multiple_of(t0, CH), CH)],
                        idxn_v.at[q])
        pltpu.sync_copy(n0_hbm.at[pl.ds(pl.multiple_of(t0 * NS0, G0), G0)],
                        idx0a_v.at[q])
        pltpu.sync_copy(n1_hbm.at[pl.ds(pl.multiple_of(t0 * NS0 * NS1, G1), G1)],
                        idx1_v.at[q])

        def remap_n(i, _):
            idxn_v[q, pl.ds(i * L, L)] = idxn_v[q, pl.ds(i * L, L)] * 4
            return 0

        def remap_0(i, _):
            v = idx0a_v[q, pl.ds(i * L, L)] * 4
            idx0n_v[q, pl.ds(i * L, L)] = v + 1
            idx0a_v[q, pl.ds(i * L, L)] = v
            return 0

        def remap_1(i, _):
            idx1_v[q, pl.ds(i * L, L)] = idx1_v[q, pl.ds(i * L, L)] * 4 + 1
            return 0

        lax.fori_loop(0, CH // L, remap_n, 0)
        lax.fori_loop(0, G0 // L, remap_0, 0)
        lax.fori_loop(0, G1 // L, remap_1, 0)

        pltpu.async_copy(t_hbm.at[idxn_v.at[q]], asel_v.at[q], sem.at[q])
        for g in range(G0 // GSUB):
            sl = pl.ds(g * GSUB, GSUB)
            pltpu.async_copy(t_hbm.at[idx0a_v.at[q, sl]], a0_v.at[q, sl],
                             sem.at[q])
            pltpu.async_copy(t_hbm.at[idx0n_v.at[q, sl]], n0_v.at[q, sl],
                             sem.at[q])
        for g in range(G1 // GSUB):
            sl = pl.ds(g * GSUB, GSUB)
            pltpu.async_copy(t_hbm.at[idx1_v.at[q, sl]], n1_v.at[q, sl],
                             sem.at[q])

    def drain(p):
        pltpu.make_async_copy(t_hbm.at[pl.ds(0, CH)], asel_v.at[p],
                              sem.at[p]).wait()
        pltpu.make_async_copy(t_hbm.at[pl.ds(0, G0)], a0_v.at[p],
                              sem.at[p]).wait()
        pltpu.make_async_copy(t_hbm.at[pl.ds(0, G0)], n0_v.at[p],
                              sem.at[p]).wait()
        pltpu.make_async_copy(t_hbm.at[pl.ds(0, G1)], n1_v.at[p],
                              sem.at[p]).wait()

    def compute(ch, p):
        @plsc.parallel_loop(0, CH, unroll=2)
        def t_body(t):
            o = t * OUT_CH
            base = t * NS0
            hs_v[pl.ds(o, L)] = jnp.maximum(asel_v[p, t, pl.ds(0, L)] + b0a, 0.0)
            hs_v[pl.ds(o + L, L)] = jnp.maximum(asel_v[p, t, pl.ds(L, L)] + b0b, 0.0)
            m0a = _tsum([n0_v[p, base + j, pl.ds(0, L)] for j in range(NS0)])
            m0b = _tsum([n0_v[p, base + j, pl.ds(L, L)] for j in range(NS0)])
            hs_v[pl.ds(o + 2 * L, L)] = jnp.maximum(m0a * (1.0 / NS0) + b0c, 0.0)
            hs_v[pl.ds(o + 3 * L, L)] = jnp.maximum(m0b * (1.0 / NS0) + b0d, 0.0)

            h1a, h1b, h1c, h1d = [], [], [], []
            for j in range(NS0):
                r = base + j
                ra = r * NS1
                m1a = _tsum([n1_v[p, ra + k, pl.ds(0, L)] for k in range(NS1)])
                m1b = _tsum([n1_v[p, ra + k, pl.ds(L, L)] for k in range(NS1)])
                h1a.append(jnp.maximum(a0_v[p, r, pl.ds(0, L)] + b0a, 0.0))
                h1b.append(jnp.maximum(a0_v[p, r, pl.ds(L, L)] + b0b, 0.0))
                h1c.append(jnp.maximum(m1a * (1.0 / NS1) + b0c, 0.0))
                h1d.append(jnp.maximum(m1b * (1.0 / NS1) + b0d, 0.0))
            hs_v[pl.ds(o + 4 * L, L)] = _tsum(h1a) * (1.0 / NS0)
            hs_v[pl.ds(o + 5 * L, L)] = _tsum(h1b) * (1.0 / NS0)
            hs_v[pl.ds(o + 6 * L, L)] = _tsum(h1c) * (1.0 / NS0)
            hs_v[pl.ds(o + 7 * L, L)] = _tsum(h1d) * (1.0 / NS0)

        t0 = wid * TW + ch * CH
        pltpu.sync_copy(
            hs_v,
            hs_hbm.at[pl.ds(pl.multiple_of(t0 * OUT_CH, CH * OUT_CH), CH * OUT_CH)])

    stage_fire(0, 0)

    def loop_body(ch, _):
        p = lax.rem(ch, 2)
        q = 1 - p

        @pl.when(ch + 1 < NCHUNK)
        def _():
            stage_fire(ch + 1, q)

        drain(p)
        compute(ch, p)
        return 0

    lax.fori_loop(0, NCHUNK, loop_body, 0)


_sc_gather = pl.kernel(
    _sc_body,
    out_type=jax.ShapeDtypeStruct((B * OUT_CH,), jnp.float32),
    mesh=plsc.VectorSubcoreMesh(
        core_axis_name="c", subcore_axis_name="s",
        num_cores=NC, num_subcores=NSUB),
    scratch_types=[
        pltpu.VMEM((2, CH), jnp.int32),
        pltpu.VMEM((2, G0), jnp.int32),
        pltpu.VMEM((2, G0), jnp.int32),
        pltpu.VMEM((2, G1), jnp.int32),
        pltpu.VMEM((2, CH, HID), jnp.float32),
        pltpu.VMEM((2, G0, HID), jnp.float32),
        pltpu.VMEM((2, G0, HID), jnp.float32),
        pltpu.VMEM((2, G1, HID), jnp.float32),
        pltpu.VMEM((4 * L,), jnp.float32),
        pltpu.VMEM((CH * OUT_CH,), jnp.float32),
        pltpu.SemaphoreType.DMA((2,)),
    ],
    compiler_params=pltpu.CompilerParams(use_tc_tiling_on_sc=False),
)


def kernel(x, nodes, neighbors_0, neighbors_1, Ws0, Wn0, b0, Ws1, Wn1, b1):
    assert x.shape == (N_NODES, IN_CH) and nodes.shape == (B,)
    assert neighbors_0.shape == (B * NS0,)
    assert neighbors_1.shape == (B * NS0 * NS1,)

    w4 = jnp.concatenate([Ws0, Wn0, Ws0, Wn0], axis=1)
    rows = 4000
    p_tab = pl.pallas_call(
        _proj_body,
        grid=(N_NODES // rows,),
        in_specs=[
            pl.BlockSpec((rows, IN_CH), lambda i: (i, 0)),
            pl.BlockSpec((IN_CH, 4 * HID), lambda i: (0, 0)),
        ],
        out_specs=pl.BlockSpec((rows, 4 * HID), lambda i: (i, 0)),
        out_shape=jax.ShapeDtypeStruct((N_NODES, 4 * HID), jnp.float32),
    )(x, w4)
    t_tab = p_tab.reshape(4 * N_NODES, HID)

    hs = _sc_gather(t_tab, nodes, neighbors_0, neighbors_1, b0)
    hs = hs.reshape(B, OUT_CH)

    w1 = jnp.concatenate([Ws1, Wn1], axis=0)
    rows2 = 1024
    out = pl.pallas_call(
        _out_body,
        grid=(B // rows2,),
        in_specs=[
            pl.BlockSpec((rows2, OUT_CH), lambda i: (i, 0)),
            pl.BlockSpec((4 * HID, OUT_CH), lambda i: (0, 0)),
            pl.BlockSpec((1, OUT_CH), lambda i: (0, 0)),
        ],
        out_specs=pl.BlockSpec((rows2, OUT_CH), lambda i: (i, 0)),
        out_shape=jax.ShapeDtypeStruct((B, OUT_CH), jnp.float32),
    )(hs, w1, b1.reshape(1, OUT_CH))
    return out


# speedup vs baseline: 1.4409x; 1.4409x over previous
import functools

import jax
import jax.numpy as jnp
from jax import lax
from jax.experimental import pallas as pl
from jax.experimental.pallas import tpu as pltpu
from jax.experimental.pallas import tpu_sc as plsc

N_NODES = 100000
IN_CH = 128
HID = 32
OUT_CH = 128
B = 8192
NS0 = 15
NS1 = 5

NC = 2
NSUB = 16
NW = NC * NSUB
TW = B // NW
CH = 8
NCHUNK = TW // CH
G0 = CH * NS0
G1 = CH * NS0 * NS1
GSUB = 120
G0T = TW * NS0
G1T = TW * NS0 * NS1

L = 16


def _tsum(vs):
    while len(vs) > 1:
        vs = [vs[i] + vs[i + 1] if i + 1 < len(vs) else vs[i]
              for i in range(0, len(vs), 2)]
    return vs[0]


def _proj_body(x_ref, w_ref, p_ref):
    p_ref[...] = jnp.dot(x_ref[...], w_ref[...],
                         preferred_element_type=jnp.float32)


def _out_body(hs_ref, w_ref, b_ref, o_ref):
    o_ref[...] = (
        jnp.dot(hs_ref[...], w_ref[...], preferred_element_type=jnp.float32)
        + b_ref[...]
    )


def _sc_body(t_hbm, nodes_hbm, n0_hbm, n1_hbm, b0_hbm, hs_hbm,
             idxn_v, idx0a_v, idx0n_v, idx1_v, asel_v, a0_v, n0_v, n1_v,
             b0_v, hs_v, sem):
    wid = lax.axis_index("s") * NC + lax.axis_index("c")

    pltpu.sync_copy(b0_hbm, b0_v)
    b0a = b0_v[pl.ds(0, L)]
    b0b = b0_v[pl.ds(L, L)]
    b0c = b0_v[pl.ds(2 * L, L)]
    b0d = b0_v[pl.ds(3 * L, L)]
    zero = jnp.zeros((L,), jnp.float32)

    def stage_all():
        t0 = wid * TW
        pltpu.sync_copy(nodes_hbm.at[pl.ds(pl.multiple_of(t0, TW), TW)], idxn_v)
        pltpu.sync_copy(n0_hbm.at[pl.ds(pl.multiple_of(t0 * NS0, G0T), G0T)],
                        idx0a_v)
        pltpu.sync_copy(n1_hbm.at[pl.ds(pl.multiple_of(t0 * NS0 * NS1, G1T), G1T)],
                        idx1_v)

        @plsc.parallel_loop(0, TW // L, unroll=2)
        def remap_n(i):
            idxn_v[pl.ds(i * L, L)] = idxn_v[pl.ds(i * L, L)] * 4

        @plsc.parallel_loop(0, G0T // L, unroll=4)
        def remap_0(i):
            v = idx0a_v[pl.ds(i * L, L)] * 4
            idx0n_v[pl.ds(i * L, L)] = v + 1
            idx0a_v[pl.ds(i * L, L)] = v

        @plsc.parallel_loop(0, G1T // L, unroll=4)
        def remap_1(i):
            idx1_v[pl.ds(i * L, L)] = idx1_v[pl.ds(i * L, L)] * 4 + 1

    def fire(ch, q):
        sln = pl.ds(pl.multiple_of(ch * CH, CH), CH)
        pltpu.async_copy(t_hbm.at[idxn_v.at[sln]], asel_v.at[q], sem.at[q])
        sl0 = pl.ds(pl.multiple_of(ch * G0, G0), G0)
        pltpu.async_copy(t_hbm.at[idx0a_v.at[sl0]], a0_v.at[q], sem.at[q])
        pltpu.async_copy(t_hbm.at[idx0n_v.at[sl0]], n0_v.at[q], sem.at[q])
        for g in range(G1 // GSUB):
            sl = pl.ds(pl.multiple_of(ch * G1, G1) + g * GSUB, GSUB)
            dst = pl.ds(g * GSUB, GSUB)
            pltpu.async_copy(t_hbm.at[idx1_v.at[sl]], n1_v.at[q, dst],
                             sem.at[q])

    def drain(p):
        pltpu.make_async_copy(t_hbm.at[pl.ds(0, CH)], asel_v.at[p],
                              sem.at[p]).wait()
        pltpu.make_async_copy(t_hbm.at[pl.ds(0, G0)], a0_v.at[p],
                              sem.at[p]).wait()
        pltpu.make_async_copy(t_hbm.at[pl.ds(0, G0)], n0_v.at[p],
                              sem.at[p]).wait()
        pltpu.make_async_copy(t_hbm.at[pl.ds(0, G1)], n1_v.at[p],
                              sem.at[p]).wait()

    def compute(ch, p):
        @plsc.parallel_loop(0, CH, unroll=2)
        def t_body(t):
            o = t * OUT_CH
            hs_v[pl.ds(o, L)] = jnp.maximum(asel_v[p, t, pl.ds(0, L)] + b0a, 0.0)
            hs_v[pl.ds(o + L, L)] = jnp.maximum(asel_v[p, t, pl.ds(L, L)] + b0b, 0.0)

            def r15(j, acc):
                r = t * NS0 + j
                ra = r * NS1
                m1a = (n1_v[p, ra, pl.ds(0, L)] + n1_v[p, ra + 1, pl.ds(0, L)]
                       + n1_v[p, ra + 2, pl.ds(0, L)] + n1_v[p, ra + 3, pl.ds(0, L)]
                       + n1_v[p, ra + 4, pl.ds(0, L)])
                m1b = (n1_v[p, ra, pl.ds(L, L)] + n1_v[p, ra + 1, pl.ds(L, L)]
                       + n1_v[p, ra + 2, pl.ds(L, L)] + n1_v[p, ra + 3, pl.ds(L, L)]
                       + n1_v[p, ra + 4, pl.ds(L, L)])
                return (acc[0] + n0_v[p, r, pl.ds(0, L)],
                        acc[1] + n0_v[p, r, pl.ds(L, L)],
                        acc[2] + jnp.maximum(a0_v[p, r, pl.ds(0, L)] + b0a, 0.0),
                        acc[3] + jnp.maximum(a0_v[p, r, pl.ds(L, L)] + b0b, 0.0),
                        acc[4] + jnp.maximum(m1a * (1.0 / NS1) + b0c, 0.0),
                        acc[5] + jnp.maximum(m1b * (1.0 / NS1) + b0d, 0.0))

            m0a, m0b, s0, s1, s2, s3 = lax.fori_loop(
                0, NS0, r15, (zero, zero, zero, zero, zero, zero))
            hs_v[pl.ds(o + 2 * L, L)] = jnp.maximum(m0a * (1.0 / NS0) + b0c, 0.0)
            hs_v[pl.ds(o + 3 * L, L)] = jnp.maximum(m0b * (1.0 / NS0) + b0d, 0.0)
            hs_v[pl.ds(o + 4 * L, L)] = s0 * (1.0 / NS0)
            hs_v[pl.ds(o + 5 * L, L)] = s1 * (1.0 / NS0)
            hs_v[pl.ds(o + 6 * L, L)] = s2 * (1.0 / NS0)
            hs_v[pl.ds(o + 7 * L, L)] = s3 * (1.0 / NS0)

        t0 = wid * TW + ch * CH
        pltpu.sync_copy(
            hs_v,
            hs_hbm.at[pl.ds(pl.multiple_of(t0 * OUT_CH, CH * OUT_CH), CH * OUT_CH)])

    stage_all()
    fire(0, 0)

    def loop_body(ch, _):
        p = lax.rem(ch, 2)
        q = 1 - p

        @pl.when(ch + 1 < NCHUNK)
        def _():
            fire(ch + 1, q)

        drain(p)
        compute(ch, p)
        return 0

    lax.fori_loop(0, NCHUNK, loop_body, 0)


_sc_gather = pl.kernel(
    _sc_body,
    out_type=jax.ShapeDtypeStruct((B * OUT_CH,), jnp.float32),
    mesh=plsc.VectorSubcoreMesh(
        core_axis_name="c", subcore_axis_name="s",
        num_cores=NC, num_subcores=NSUB),
    scratch_types=[
        pltpu.VMEM((TW,), jnp.int32),
        pltpu.VMEM((G0T,), jnp.int32),
        pltpu.VMEM((G0T,), jnp.int32),
        pltpu.VMEM((G1T,), jnp.int32),
        pltpu.VMEM((2, CH, HID), jnp.float32),
        pltpu.VMEM((2, G0, HID), jnp.float32),
        pltpu.VMEM((2, G0, HID), jnp.float32),
        pltpu.VMEM((2, G1, HID), jnp.float32),
        pltpu.VMEM((4 * L,), jnp.float32),
        pltpu.VMEM((CH * OUT_CH,), jnp.float32),
        pltpu.SemaphoreType.DMA((2,)),
    ],
    compiler_params=pltpu.CompilerParams(use_tc_tiling_on_sc=False),
)


def kernel(x, nodes, neighbors_0, neighbors_1, Ws0, Wn0, b0, Ws1, Wn1, b1):
    assert x.shape == (N_NODES, IN_CH) and nodes.shape == (B,)
    assert neighbors_0.shape == (B * NS0,)
    assert neighbors_1.shape == (B * NS0 * NS1,)

    w4 = jnp.concatenate([Ws0, Wn0, Ws0, Wn0], axis=1)
    rows = 4000
    p_tab = pl.pallas_call(
        _proj_body,
        grid=(N_NODES // rows,),
        in_specs=[
            pl.BlockSpec((rows, IN_CH), lambda i: (i, 0)),
            pl.BlockSpec((IN_CH, 4 * HID), lambda i: (0, 0)),
        ],
        out_specs=pl.BlockSpec((rows, 4 * HID), lambda i: (i, 0)),
        out_shape=jax.ShapeDtypeStruct((N_NODES, 4 * HID), jnp.float32),
    )(x, w4)
    t_tab = p_tab.reshape(4 * N_NODES, HID)

    hs = _sc_gather(t_tab, nodes, neighbors_0, neighbors_1, b0)
    hs = hs.reshape(B, OUT_CH)

    w1 = jnp.concatenate([Ws1, Wn1], axis=0)
    rows2 = 1024
    out = pl.pallas_call(
        _out_body,
        grid=(B // rows2,),
        in_specs=[
            pl.BlockSpec((rows2, OUT_CH), lambda i: (i, 0)),
            pl.BlockSpec((4 * HID, OUT_CH), lambda i: (0, 0)),
            pl.BlockSpec((1, OUT_CH), lambda i: (0, 0)),
        ],
        out_specs=pl.BlockSpec((rows2, OUT_CH), lambda i: (i, 0)),
        out_shape=jax.ShapeDtypeStruct((B, OUT_CH), jnp.float32),
    )(hs, w1, b1.reshape(1, OUT_CH))
    return out


# speedup vs baseline: 1.5156x; 1.0519x over previous
import functools

import jax
import jax.numpy as jnp
from jax import lax
from jax.experimental import pallas as pl
from jax.experimental.pallas import tpu as pltpu
from jax.experimental.pallas import tpu_sc as plsc

N_NODES = 100000
IN_CH = 128
HID = 32
OUT_CH = 128
B = 8192
NS0 = 15
NS1 = 5

NC = 2
NSUB = 16
NW = NC * NSUB
TW = B // NW
CH = 8
NCHUNK = TW // CH
G0 = CH * NS0
G1 = CH * NS0 * NS1
GSUB = 120
G0T = TW * NS0
G1T = TW * NS0 * NS1

L = 16


def _tsum(vs):
    while len(vs) > 1:
        vs = [vs[i] + vs[i + 1] if i + 1 < len(vs) else vs[i]
              for i in range(0, len(vs), 2)]
    return vs[0]


def _proj_body(x_ref, w_ref, p_ref):
    p_ref[...] = jnp.dot(x_ref[...], w_ref[...],
                         preferred_element_type=jnp.float32)


def _out_body(hs_ref, w_ref, b_ref, o_ref):
    o_ref[...] = (
        jnp.dot(hs_ref[...], w_ref[...], preferred_element_type=jnp.float32)
        + b_ref[...]
    )


def _sc_body(t_hbm, nodes_hbm, n0_hbm, n1_hbm, b0_hbm, hs_hbm,
             idxn_v, idx0a_v, idx0n_v, idx1_v, asel_v, a0_v, n0_v, n1_v,
             b0_v, hs_v, sem):
    wid = lax.axis_index("s") * NC + lax.axis_index("c")

    pltpu.sync_copy(b0_hbm, b0_v)
    b0a = b0_v[pl.ds(0, L)]
    b0b = b0_v[pl.ds(L, L)]
    b0c = b0_v[pl.ds(2 * L, L)]
    b0d = b0_v[pl.ds(3 * L, L)]
    zero = jnp.zeros((L,), jnp.float32)

    def stage_all():
        t0 = wid * TW
        pltpu.sync_copy(nodes_hbm.at[pl.ds(pl.multiple_of(t0, TW), TW)], idxn_v)
        pltpu.sync_copy(n0_hbm.at[pl.ds(pl.multiple_of(t0 * NS0, G0T), G0T)],
                        idx0a_v)
        pltpu.sync_copy(n1_hbm.at[pl.ds(pl.multiple_of(t0 * NS0 * NS1, G1T), G1T)],
                        idx1_v)

        @plsc.parallel_loop(0, TW // L, unroll=2)
        def remap_n(i):
            idxn_v[pl.ds(i * L, L)] = idxn_v[pl.ds(i * L, L)] * 4

        @plsc.parallel_loop(0, G0T // L, unroll=4)
        def remap_0(i):
            v = idx0a_v[pl.ds(i * L, L)] * 4
            idx0n_v[pl.ds(i * L, L)] = v + 1
            idx0a_v[pl.ds(i * L, L)] = v

        @plsc.parallel_loop(0, G1T // L, unroll=4)
        def remap_1(i):
            idx1_v[pl.ds(i * L, L)] = idx1_v[pl.ds(i * L, L)] * 4 + 1

    def fire(ch, q):
        sln = pl.ds(pl.multiple_of(ch * CH, CH), CH)
        pltpu.async_copy(t_hbm.at[idxn_v.at[sln]], asel_v.at[q], sem.at[q])
        sl0 = pl.ds(pl.multiple_of(ch * G0, G0), G0)
        pltpu.async_copy(t_hbm.at[idx0a_v.at[sl0]], a0_v.at[q], sem.at[q])
        pltpu.async_copy(t_hbm.at[idx0n_v.at[sl0]], n0_v.at[q], sem.at[q])
        for g in range(G1 // GSUB):
            sl = pl.ds(pl.multiple_of(ch * G1, G1) + g * GSUB, GSUB)
            dst = pl.ds(g * GSUB, GSUB)
            pltpu.async_copy(t_hbm.at[idx1_v.at[sl]], n1_v.at[q, dst],
                             sem.at[q])

    def drain(p):
        pltpu.make_async_copy(t_hbm.at[pl.ds(0, CH)], asel_v.at[p],
                              sem.at[p]).wait()
        pltpu.make_async_copy(t_hbm.at[pl.ds(0, G0)], a0_v.at[p],
                              sem.at[p]).wait()
        pltpu.make_async_copy(t_hbm.at[pl.ds(0, G0)], n0_v.at[p],
                              sem.at[p]).wait()
        pltpu.make_async_copy(t_hbm.at[pl.ds(0, G1)], n1_v.at[p],
                              sem.at[p]).wait()

    def compute(ch, p):
        @plsc.parallel_loop(0, CH, unroll=2)
        def t_body(t):
            o = t * OUT_CH
            hs_v[pl.ds(o, L)] = jnp.maximum(asel_v[p, t, pl.ds(0, L)] + b0a, 0.0)
            hs_v[pl.ds(o + L, L)] = jnp.maximum(asel_v[p, t, pl.ds(L, L)] + b0b, 0.0)

            def r15(j, acc):
                r = t * NS0 + j
                ra = r * NS1
                m1a = (n1_v[p, ra, pl.ds(0, L)] + n1_v[p, ra + 1, pl.ds(0, L)]
                       + n1_v[p, ra + 2, pl.ds(0, L)] + n1_v[p, ra + 3, pl.ds(0, L)]
                       + n1_v[p, ra + 4, pl.ds(0, L)])
                m1b = (n1_v[p, ra, pl.ds(L, L)] + n1_v[p, ra + 1, pl.ds(L, L)]
                       + n1_v[p, ra + 2, pl.ds(L, L)] + n1_v[p, ra + 3, pl.ds(L, L)]
                       + n1_v[p, ra + 4, pl.ds(L, L)])
                return (acc[0] + n0_v[p, r, pl.ds(0, L)],
                        acc[1] + n0_v[p, r, pl.ds(L, L)],
                        acc[2] + jnp.maximum(a0_v[p, r, pl.ds(0, L)] + b0a, 0.0),
                        acc[3] + jnp.maximum(a0_v[p, r, pl.ds(L, L)] + b0b, 0.0),
                        acc[4] + jnp.maximum(m1a * (1.0 / NS1) + b0c, 0.0),
                        acc[5] + jnp.maximum(m1b * (1.0 / NS1) + b0d, 0.0))

            m0a, m0b, s0, s1, s2, s3 = lax.fori_loop(
                0, NS0, r15, (zero, zero, zero, zero, zero, zero))
            hs_v[pl.ds(o + 2 * L, L)] = jnp.maximum(m0a * (1.0 / NS0) + b0c, 0.0)
            hs_v[pl.ds(o + 3 * L, L)] = jnp.maximum(m0b * (1.0 / NS0) + b0d, 0.0)
            hs_v[pl.ds(o + 4 * L, L)] = s0 * (1.0 / NS0)
            hs_v[pl.ds(o + 5 * L, L)] = s1 * (1.0 / NS0)
            hs_v[pl.ds(o + 6 * L, L)] = s2 * (1.0 / NS0)
            hs_v[pl.ds(o + 7 * L, L)] = s3 * (1.0 / NS0)

        t0 = wid * TW + ch * CH
        pltpu.sync_copy(
            hs_v,
            hs_hbm.at[pl.ds(pl.multiple_of(t0 * OUT_CH, CH * OUT_CH), CH * OUT_CH)])

    stage_all()
    fire(0, 0)

    def loop_body(ch, _):
        p = lax.rem(ch, 2)
        q = 1 - p

        @pl.when(ch + 1 < NCHUNK)
        def _():
            fire(ch + 1, q)

        drain(p)
        compute(ch, p)
        return 0

    lax.fori_loop(0, NCHUNK, loop_body, 0)


_sc_gather = pl.kernel(
    _sc_body,
    out_type=jax.ShapeDtypeStruct((B * OUT_CH,), jnp.float32),
    mesh=plsc.VectorSubcoreMesh(
        core_axis_name="c", subcore_axis_name="s",
        num_cores=NC, num_subcores=NSUB),
    scratch_types=[
        pltpu.VMEM((TW,), jnp.int32),
        pltpu.VMEM((G0T,), jnp.int32),
        pltpu.VMEM((G0T,), jnp.int32),
        pltpu.VMEM((G1T,), jnp.int32),
        pltpu.VMEM((2, CH, HID), jnp.float32),
        pltpu.VMEM((2, G0, HID), jnp.float32),
        pltpu.VMEM((2, G0, HID), jnp.float32),
        pltpu.VMEM((2, G1, HID), jnp.float32),
        pltpu.VMEM((4 * L,), jnp.float32),
        pltpu.VMEM((CH * OUT_CH,), jnp.float32),
        pltpu.SemaphoreType.DMA((2,)),
    ],
    compiler_params=pltpu.CompilerParams(use_tc_tiling_on_sc=False),
)


def kernel(x, nodes, neighbors_0, neighbors_1, Ws0, Wn0, b0, Ws1, Wn1, b1):
    assert x.shape == (N_NODES, IN_CH) and nodes.shape == (B,)
    assert neighbors_0.shape == (B * NS0,)
    assert neighbors_1.shape == (B * NS0 * NS1,)

    w4 = jnp.concatenate([Ws0, Wn0, Ws0, Wn0], axis=1)
    rows = 10000
    p_tab = pl.pallas_call(
        _proj_body,
        grid=(N_NODES // rows,),
        in_specs=[
            pl.BlockSpec((rows, IN_CH), lambda i: (i, 0)),
            pl.BlockSpec((IN_CH, 4 * HID), lambda i: (0, 0)),
        ],
        out_specs=pl.BlockSpec((rows, 4 * HID), lambda i: (i, 0)),
        out_shape=jax.ShapeDtypeStruct((N_NODES, 4 * HID), jnp.float32),
    )(x, w4)
    t_tab = p_tab.reshape(4 * N_NODES, HID)

    hs = _sc_gather(t_tab, nodes, neighbors_0, neighbors_1, b0)
    hs = hs.reshape(B, OUT_CH)

    w1 = jnp.concatenate([Ws1, Wn1], axis=0)
    rows2 = 1024
    out = pl.pallas_call(
        _out_body,
        grid=(B // rows2,),
        in_specs=[
            pl.BlockSpec((rows2, OUT_CH), lambda i: (i, 0)),
            pl.BlockSpec((4 * HID, OUT_CH), lambda i: (0, 0)),
            pl.BlockSpec((1, OUT_CH), lambda i: (0, 0)),
        ],
        out_specs=pl.BlockSpec((rows2, OUT_CH), lambda i: (i, 0)),
        out_shape=jax.ShapeDtypeStruct((B, OUT_CH), jnp.float32),
    )(hs, w1, b1.reshape(1, OUT_CH))
    return out
